# baseline (speedup 1.0000x reference)
import jax
import jax.numpy as jnp
from jax import lax
from jax.experimental import pallas as pl
from jax.experimental.pallas import tpu as pltpu

DIM = 1024
AUG = DIM + 16
BLK = 512

_DN_TT = (((0,), (1,)), ((), ()))
_DN_NN = (((1,), (0,)), ((), ()))


def _body(w3a, w4, w1a, w2, b4, b2, x, xa, ya, out, s23):
    i = pl.program_id(0)

    @pl.when(i == 0)
    def _combine():
        W3a = w3a[...].astype(jnp.bfloat16)
        W4 = w4[...].astype(jnp.bfloat16)
        W1a = w1a[...].astype(jnp.bfloat16)
        W2 = w2[...].astype(jnp.bfloat16)
        rowmask = (lax.broadcasted_iota(jnp.int32, (AUG, DIM), 0) == DIM
                   ).astype(jnp.float32)
        S2 = lax.dot_general(W3a, W4, _DN_TT,
                             preferred_element_type=jnp.float32)
        S2 = S2 + rowmask * b4[...]
        S1 = lax.dot_general(W1a, W2, _DN_TT,
                             preferred_element_type=jnp.float32)
        S1 = S1 + rowmask * b2[...]
        S2b = S2.astype(jnp.bfloat16)
        s23[:, :DIM] = S2b
        S3 = lax.dot_general(S1.astype(jnp.bfloat16), S2b[:DIM, :], _DN_NN,
                             preferred_element_type=jnp.float32)
        S3 = S3 + rowmask * S2[DIM:DIM + 1, :]
        s23[:, DIM:] = S3.astype(jnp.bfloat16)

    @pl.when(i > 0)
    def _apply():
        xb = x[...]
        xh = xb.astype(jnp.bfloat16)
        Y = lax.dot_general(xh, s23[:DIM, :], _DN_NN,
                            preferred_element_type=jnp.float32)
        Y = Y + s23[DIM:DIM + 1, :].astype(jnp.float32)
        y2 = Y[:, :DIM]
        y3 = Y[:, DIM:]

        def level(a):
            a0, a1, a2 = a[:, 0:1], a[:, 1:2], a[:, 2:3]
            return jnp.where((a0 >= a1) & (a0 >= a2), 0,
                             jnp.where(a1 >= a2, 1, 2))

        lvl = jnp.maximum(level(xa[...]), level(ya[...]))
        out[...] = jnp.where(lvl == 2, y3, jnp.where(lvl == 1, y2, xb))


def _augment(W, b):
    return jnp.concatenate(
        [W, b.reshape(DIM, 1),
         jnp.zeros((DIM, AUG - DIM - 1), jnp.float32)], axis=1)


def kernel(x_feat, x_angle, y_angle, W1, b1, W2, b2, W3, b3, W4, b4):
    W3a = _augment(W3, b3)
    W1a = _augment(W1, b1)
    b4r = b4.reshape(1, DIM)
    b2r = b2.reshape(1, DIM)

    n = x_feat.shape[0]
    nblk = n // BLK

    def prev(i):
        return jnp.maximum(i - 1, 0)

    out = pl.pallas_call(
        _body,
        grid=(nblk + 1,),
        in_specs=[
            pl.BlockSpec((DIM, AUG), lambda i: (0, 0)),
            pl.BlockSpec((DIM, DIM), lambda i: (0, 0)),
            pl.BlockSpec((DIM, AUG), lambda i: (0, 0)),
            pl.BlockSpec((DIM, DIM), lambda i: (0, 0)),
            pl.BlockSpec((1, DIM), lambda i: (0, 0)),
            pl.BlockSpec((1, DIM), lambda i: (0, 0)),
            pl.BlockSpec((BLK, DIM), lambda i: (prev(i), 0)),
            pl.BlockSpec((BLK, 3), lambda i: (prev(i), 0)),
            pl.BlockSpec((BLK, 3), lambda i: (prev(i), 0)),
        ],
        out_specs=pl.BlockSpec((BLK, DIM), lambda i: (prev(i), 0)),
        out_shape=jax.ShapeDtypeStruct((n, DIM), jnp.float32),
        scratch_shapes=[
            pltpu.VMEM((AUG, 2 * DIM), jnp.bfloat16),
        ],
    )(W3a, W4, W1a, W2, b4r, b2r, x_feat, x_angle, y_angle)
    return out


# speedup vs baseline: 1.0689x; 1.0689x over previous
import jax
import jax.numpy as jnp
from jax import lax
from jax.experimental import pallas as pl
from jax.experimental.pallas import tpu as pltpu

DIM = 1024
PAD = 16
BLK = 512

_DN_TT = (((0,), (1,)), ((), ()))
_DN_NN = (((1,), (0,)), ((), ()))


def _body(w3, w4, w1, w2, b3p, b1p, b4r, b2r, x, xa, ya, out, s23):
    i = pl.program_id(0)

    @pl.when(i == 0)
    def _combine():
        W3 = w3[...].astype(jnp.bfloat16)
        W4 = w4[...].astype(jnp.bfloat16)
        W1 = w1[...].astype(jnp.bfloat16)
        W2 = w2[...].astype(jnp.bfloat16)
        S2top = lax.dot_general(W3, W4, _DN_TT,
                                preferred_element_type=jnp.float32)
        S2bot = lax.dot_general(b3p[...].astype(jnp.bfloat16), W4, _DN_TT,
                                preferred_element_type=jnp.float32) + b4r[...]
        S2topb = S2top.astype(jnp.bfloat16)
        s23[:DIM, :DIM] = S2topb
        s23[DIM:, :DIM] = S2bot.astype(jnp.bfloat16)
        S1top = lax.dot_general(W1, W2, _DN_TT,
                                preferred_element_type=jnp.float32)
        S1bot = lax.dot_general(b1p[...].astype(jnp.bfloat16), W2, _DN_TT,
                                preferred_element_type=jnp.float32) + b2r[...]
        S1 = jnp.concatenate([S1top, S1bot], axis=0).astype(jnp.bfloat16)
        S3 = lax.dot_general(S1, S2topb, _DN_NN,
                             preferred_element_type=jnp.float32)
        rowmask = (lax.broadcasted_iota(jnp.int32, (DIM + PAD, DIM), 0) == DIM
                   ).astype(jnp.float32)
        S3 = S3 + rowmask * S2bot[0:1, :]
        s23[:, DIM:] = S3.astype(jnp.bfloat16)

    @pl.when(i > 0)
    def _apply():
        xb = x[...]
        xh = xb.astype(jnp.bfloat16)
        Y = lax.dot_general(xh, s23[:DIM, :], _DN_NN,
                            preferred_element_type=jnp.float32)
        Y = Y + s23[DIM:DIM + 1, :].astype(jnp.float32)
        y2 = Y[:, :DIM]
        y3 = Y[:, DIM:]

        def level(a):
            a0, a1, a2 = a[:, 0:1], a[:, 1:2], a[:, 2:3]
            return jnp.where((a0 >= a1) & (a0 >= a2), 0,
                             jnp.where(a1 >= a2, 1, 2))

        lvl = jnp.maximum(level(xa[...]), level(ya[...]))
        out[...] = jnp.where(lvl == 2, y3, jnp.where(lvl == 1, y2, xb))


def kernel(x_feat, x_angle, y_angle, W1, b1, W2, b2, W3, b3, W4, b4):
    zc = jnp.zeros((DIM, PAD - 1), jnp.float32)
    b3p = jnp.concatenate([b3.reshape(DIM, 1), zc], axis=1)
    b1p = jnp.concatenate([b1.reshape(DIM, 1), zc], axis=1)
    zr = jnp.zeros((PAD - 1, DIM), jnp.float32)
    b4r = jnp.concatenate([b4.reshape(1, DIM), zr], axis=0)
    b2r = jnp.concatenate([b2.reshape(1, DIM), zr], axis=0)

    n = x_feat.shape[0]
    nblk = n // BLK

    def prev(i):
        return jnp.maximum(i - 1, 0)

    out = pl.pallas_call(
        _body,
        grid=(nblk + 1,),
        in_specs=[
            pl.BlockSpec((DIM, DIM), lambda i: (0, 0)),
            pl.BlockSpec((DIM, DIM), lambda i: (0, 0)),
            pl.BlockSpec((DIM, DIM), lambda i: (0, 0)),
            pl.BlockSpec((DIM, DIM), lambda i: (0, 0)),
            pl.BlockSpec((DIM, PAD), lambda i: (0, 0)),
            pl.BlockSpec((DIM, PAD), lambda i: (0, 0)),
            pl.BlockSpec((PAD, DIM), lambda i: (0, 0)),
            pl.BlockSpec((PAD, DIM), lambda i: (0, 0)),
            pl.BlockSpec((BLK, DIM), lambda i: (prev(i), 0)),
            pl.BlockSpec((BLK, 3), lambda i: (prev(i), 0)),
            pl.BlockSpec((BLK, 3), lambda i: (prev(i), 0)),
        ],
        out_specs=pl.BlockSpec((BLK, DIM), lambda i: (prev(i), 0)),
        out_shape=jax.ShapeDtypeStruct((n, DIM), jnp.float32),
        scratch_shapes=[
            pltpu.VMEM((DIM + PAD, 2 * DIM), jnp.bfloat16),
        ],
    )(W3, W4, W1, W2, b3p, b1p, b4r, b2r, x_feat, x_angle, y_angle)
    return out
